# baseline (speedup 1.0000x reference)
import functools

import jax
import jax.numpy as jnp
from jax import lax
from jax.experimental import pallas as pl
from jax.experimental.pallas import tpu as pltpu
from jax.experimental.pallas import tpu_sc as plsc

B, L, D = 16, 4096, 1024
NW = 32
G = 8
NG = (B * L) // G
GPW = NG // NW
GPW_BITS = 9
ZC = 8
ZC_LOG = 3

_mesh = plsc.VectorSubcoreMesh(core_axis_name="c", subcore_axis_name="s")


@functools.partial(
    pl.kernel,
    mesh=_mesh,
    out_type=jax.ShapeDtypeStruct((NG, G, D), jnp.float32),
    scratch_types=[
        pltpu.VMEM((NW, 16), jnp.int32),
        pltpu.VMEM((ZC, G, D), jnp.float32),
        pltpu.VMEM((G, D), jnp.float32),
    ],
)
def _squeeze_sc(x_hbm, nv_hbm, z_hbm, out_hbm, nv_v, zbuf, bbuf):
    wid = lax.axis_index("s") * 2 + lax.axis_index("c")
    base = wid * GPW
    pltpu.sync_copy(nv_hbm, nv_v)
    pltpu.sync_copy(z_hbm, zbuf)
    nv = nv_v[wid][0]
    nfg = nv >> 3
    r = nv & 7

    for k in range(GPW_BITS - 1, -1, -1):
        size = 1 << k
        pos = base + ((nfg >> (k + 1)) << (k + 1))

        @pl.when((nfg & size) != 0)
        def _copy(pos=pos, size=size):
            pltpu.sync_copy(
                x_hbm.at[pl.ds(pos, size)], out_hbm.at[pl.ds(pos, size)]
            )

    gb = base + nfg

    @pl.when(r != 0)
    def _boundary():
        pltpu.sync_copy(x_hbm.at[gb], bbuf)
        zv = jnp.zeros((16,), jnp.float32)
        for row in range(1, G):

            @pl.when(row >= r)
            def _zero_row(row=row):
                def _st(c, carry):
                    bbuf[row, pl.ds(c * 16, 16)] = zv
                    return carry

                lax.fori_loop(0, D // 16, _st, 0)

        pltpu.sync_copy(bbuf, out_hbm.at[gb])

    zstart = gb + (r != 0).astype(jnp.int32)
    mg = base + GPW - zstart
    nfull = mg >> ZC_LOG

    def _zero_chunk(i, carry):
        pltpu.sync_copy(zbuf, out_hbm.at[pl.ds(zstart + (i << ZC_LOG), ZC)])
        return carry

    lax.fori_loop(0, nfull, _zero_chunk, 0)
    for k in range(ZC_LOG - 1, -1, -1):
        size = 1 << k
        zpos = zstart + ((mg >> (k + 1)) << (k + 1))

        @pl.when((mg & size) != 0)
        def _zero_rem(zpos=zpos, size=size):
            pltpu.sync_copy(zbuf.at[pl.ds(0, size)], out_hbm.at[pl.ds(zpos, size)])


def kernel(x, x_len):
    xl = x_len.astype(jnp.int32)
    off = (jnp.arange(NW, dtype=jnp.int32) % 2) * (G * GPW)
    nv = jnp.clip(jnp.repeat(xl, 2) - off, 0, G * GPW)
    nv = jnp.broadcast_to(nv[:, None], (NW, 16))
    zsrc = jnp.zeros((ZC, G, D), jnp.float32)
    out = _squeeze_sc(x.reshape(NG, G, D), nv, zsrc)
    return out.reshape(B, L, D)


# speedup vs baseline: 1.0147x; 1.0147x over previous
import functools

import jax
import jax.numpy as jnp
from jax import lax
from jax.experimental import pallas as pl
from jax.experimental.pallas import tpu as pltpu
from jax.experimental.pallas import tpu_sc as plsc

B, L, D = 16, 4096, 1024
NW = 32
G = 8
NG = (B * L) // G
GPW = NG // NW
GPW_BITS = 9
ZC = 8
ZC_LOG = 3

_mesh = plsc.VectorSubcoreMesh(core_axis_name="c", subcore_axis_name="s")


@functools.partial(
    pl.kernel,
    mesh=_mesh,
    out_type=jax.ShapeDtypeStruct((NG, G, D), jnp.float32),
    scratch_types=[
        pltpu.VMEM((NW, 16), jnp.int32),
        pltpu.VMEM((ZC, G, D), jnp.float32),
        pltpu.VMEM((G, D), jnp.float32),
        pltpu.SemaphoreType.DMA,
        pltpu.SemaphoreType.DMA,
    ],
)
def _squeeze_sc(x_hbm, nv_hbm, z_hbm, out_hbm, nv_v, zbuf, bbuf, sem, bsem):
    wid = lax.axis_index("s") * 2 + lax.axis_index("c")
    base = wid * GPW
    pltpu.sync_copy(nv_hbm, nv_v)
    pltpu.async_copy(z_hbm, zbuf, sem)
    nv = nv_v[wid][0]
    nfg = nv >> 3
    r = nv & 7

    for k in range(GPW_BITS - 1, -1, -1):
        size = 1 << k
        pos = base + ((nfg >> (k + 1)) << (k + 1))

        @pl.when((nfg & size) != 0)
        def _copy(pos=pos, size=size):
            pltpu.async_copy(
                x_hbm.at[pl.ds(pos, size)], out_hbm.at[pl.ds(pos, size)], sem
            )

    gb = base + nfg

    @pl.when(r != 0)
    def _boundary():
        pltpu.async_copy(x_hbm.at[gb], bbuf, bsem).wait()
        zv = jnp.zeros((16,), jnp.float32)
        for row in range(1, G):

            @pl.when(row >= r)
            def _zero_row(row=row):
                def _st(c, carry):
                    bbuf[row, pl.ds(c * 16, 16)] = zv
                    return carry

                lax.fori_loop(0, D // 16, _st, 0)

        pltpu.async_copy(bbuf, out_hbm.at[gb], bsem)

    zstart = gb + (r != 0).astype(jnp.int32)
    mg = base + GPW - zstart
    nfull = mg >> ZC_LOG
    pltpu.make_async_copy(z_hbm, zbuf, sem).wait()

    def _zero_chunk(i, carry):
        pltpu.async_copy(zbuf, out_hbm.at[pl.ds(zstart + (i << ZC_LOG), ZC)], sem)
        return carry

    lax.fori_loop(0, nfull, _zero_chunk, 0)
    for k in range(ZC_LOG - 1, -1, -1):
        size = 1 << k
        zpos = zstart + ((mg >> (k + 1)) << (k + 1))

        @pl.when((mg & size) != 0)
        def _zero_rem(zpos=zpos, size=size):
            pltpu.async_copy(zbuf.at[pl.ds(0, size)], out_hbm.at[pl.ds(zpos, size)], sem)

    for k in range(GPW_BITS - 1, -1, -1):
        size = 1 << k
        pos = base + ((nfg >> (k + 1)) << (k + 1))

        @pl.when((nfg & size) != 0)
        def _copy_wait(pos=pos, size=size):
            pltpu.make_async_copy(
                x_hbm.at[pl.ds(pos, size)], out_hbm.at[pl.ds(pos, size)], sem
            ).wait()

    def _zero_chunk_wait(i, carry):
        pltpu.make_async_copy(
            zbuf, out_hbm.at[pl.ds(zstart + (i << ZC_LOG), ZC)], sem
        ).wait()
        return carry

    lax.fori_loop(0, nfull, _zero_chunk_wait, 0)
    for k in range(ZC_LOG - 1, -1, -1):
        size = 1 << k
        zpos = zstart + ((mg >> (k + 1)) << (k + 1))

        @pl.when((mg & size) != 0)
        def _zero_rem_wait(zpos=zpos, size=size):
            pltpu.make_async_copy(
                zbuf.at[pl.ds(0, size)], out_hbm.at[pl.ds(zpos, size)], sem
            ).wait()

    @pl.when(r != 0)
    def _boundary_wait():
        pltpu.make_async_copy(bbuf, out_hbm.at[gb], bsem).wait()


def kernel(x, x_len):
    xl = x_len.astype(jnp.int32)
    off = (jnp.arange(NW, dtype=jnp.int32) % 2) * (G * GPW)
    nv = jnp.clip(jnp.repeat(xl, 2) - off, 0, G * GPW)
    nv = jnp.broadcast_to(nv[:, None], (NW, 16))
    zsrc = jnp.zeros((ZC, G, D), jnp.float32)
    out = _squeeze_sc(x.reshape(NG, G, D), nv, zsrc)
    return out.reshape(B, L, D)
